# Initial kernel scaffold; baseline (speedup 1.0000x reference)
import functools
import math

import numpy as np
import jax
import jax.numpy as jnp
from jax.experimental import pallas as pl
from jax.experimental.pallas import tpu as pltpu

NH = 12
HD = 64
C = 768
SEL = [16, 14, 12, 10, 8, 6, 8, 10, 12, 14, 16]
_NEG = -1e30


def _conv_matrix() -> np.ndarray:
    hs = 14
    k = np.array([[1.0, 2.0, 1.0], [2.0, 4.0, 2.0], [1.0, 2.0, 1.0]], np.float32)
    m = np.zeros((hs * hs, hs * hs), np.float32)
    for r in range(hs):
        for c in range(hs):
            p = r * hs + c
            for dr in (-1, 0, 1):
                for dc in (-1, 0, 1):
                    rr, cc = r + dr, c + dc
                    if 0 <= rr < hs and 0 <= cc < hs:
                        m[rr * hs + cc, p] += k[dr + 1, dc + 1]
    return m


_CONV_M = _conv_matrix()


def _lnv(x, s, b, eps=1e-6):
    mu = jnp.mean(x, axis=-1, keepdims=True)
    v = jnp.mean((x - mu) ** 2, axis=-1, keepdims=True)
    return (x - mu) / jnp.sqrt(v + eps) * s + b


def _dot(a, b):
    return jnp.dot(a, b, preferred_element_type=jnp.float32)


def _dot_t(a, b):
    return jax.lax.dot_general(
        a, b, (((1,), (1,)), ((), ())), preferred_element_type=jnp.float32)


def _iter_topk(scores, kk):
    r, p = scores.shape
    lane = jax.lax.broadcasted_iota(jnp.int32, (r, p), 1)
    work = scores
    mask = jnp.zeros((r, p), jnp.float32)
    onehots = []
    for _ in range(kk):
        m = jnp.max(work, axis=1, keepdims=True)
        ismax = work >= m
        idx = jnp.min(jnp.where(ismax, lane, p), axis=1, keepdims=True)
        oh = lane == idx
        ohf = oh.astype(jnp.float32)
        mask = mask + ohf
        onehots.append(ohf)
        work = jnp.where(oh, _NEG, work)
    return mask, onehots


def _attn_mlp(h, ln1s, ln1b, wq, bq, wk, bk, wv, bv, wo, bo, ln2s, ln2b,
              w1, b1, w2, b2):
    n = h.shape[0]
    scale = 1.0 / math.sqrt(HD)
    y = _lnv(h, ln1s, ln1b)
    q = _dot(y, wq) + bq
    k = _dot(y, wk) + bk
    v = _dot(y, wv) + bv
    ctx_cols = []
    score_rows = []
    for hh in range(NH):
        sl = slice(hh * HD, (hh + 1) * HD)
        s = _dot_t(q[:, sl], k[:, sl]) * scale
        a = jax.nn.softmax(s, axis=-1)
        score_rows.append(a[0:1, 1:n])
        ctx_cols.append(_dot(a, v[:, sl]))
    ctx = jnp.concatenate(ctx_cols, axis=1)
    h1 = h + _dot(ctx, wo) + bo
    y2 = _lnv(h1, ln2s, ln2b)
    g = jax.nn.gelu(_dot(y2, w1) + b1)
    return h1 + _dot(g, w2) + b2, score_rows


def _vote_counts(score_rows_by_b, p):
    s_all = jnp.concatenate([r for rows in score_rows_by_b for r in rows], axis=0)
    mask, _ = _iter_topk(s_all, 24)
    counts = []
    for b in range(len(score_rows_by_b)):
        counts.append(jnp.sum(mask[b * NH:(b + 1) * NH, :], axis=0, keepdims=True))
    return jnp.concatenate(counts, axis=0)


def _ranked_gather(counts, kk, srcs):
    _, ohs = _iter_topk(counts, kk)
    out = []
    for b in range(counts.shape[0]):
        g = jnp.concatenate([oh[b:b + 1, :] for oh in ohs], axis=0)
        out.append(_dot(g, srcs[b]))
    return out


def _embed_body(p_ref, wp_ref, bp_ref, cls_ref, pos_ref, out_ref):
    for b in range(2):
        hp = _dot(p_ref[b], wp_ref[...]) + bp_ref[...]
        out_ref[b, 0:1, :] = cls_ref[...] + pos_ref[0:1, :]
        out_ref[b, 1:, :] = hp + pos_ref[1:, :]


def _block_body(h_ref, ln1s, ln1b, wq, bq, wk, bk, wv, bv, wo, bo,
                ln2s, ln2b, w1, b1, w2, b2, convm, hout_ref, gath_ref):
    n = h_ref.shape[1]
    rows_by_b = []
    hns = []
    for b in range(2):
        hn, rows = _attn_mlp(
            h_ref[b], ln1s[...], ln1b[...], wq[...], bq[...], wk[...], bk[...],
            wv[...], bv[...], wo[...], bo[...], ln2s[...], ln2b[...],
            w1[...], b1[...], w2[...], b2[...])
        hout_ref[b] = hn
        hns.append(hn)
        rows_by_b.append(rows)
    counts = _vote_counts(rows_by_b, n - 1)
    counts = _dot(counts, convm[...])
    gath = _ranked_gather(counts, 16, [hn[1:, :] for hn in hns])
    for b in range(2):
        gath_ref[b] = gath[b]


def _clr_body(h_ref, cls_ref, ln1s, ln1b, wq, bq, wk, bk, wv, bv, wo, bo,
              ln2s, ln2b, w1, b1, w2, b2, cns, cnb, out_ref, xc_ref):
    n = h_ref.shape[1]
    rows_by_b = []
    clrs = []
    for b in range(2):
        hn, rows = _attn_mlp(
            h_ref[b], ln1s[...], ln1b[...], wq[...], bq[...], wk[...], bk[...],
            wv[...], bv[...], wo[...], bo[...], ln2s[...], ln2b[...],
            w1[...], b1[...], w2[...], b2[...])
        clr = _lnv(hn, cns[...], cnb[...])
        clrs.append(clr)
        rows_by_b.append(rows)
        xc_ref[b] = clr[0:1, :]
    counts = _vote_counts(rows_by_b, n - 1)
    gath = _ranked_gather(counts, 24, [clr[1:, :] for clr in clrs])
    for b in range(2):
        out_ref[b, 0:1, :] = cls_ref[b]
        out_ref[b, 1:, :] = gath[b]


def _key_body(h_ref, xc_ref, ln1s, ln1b, wq, bq, wk, bk, wv, bv, wo, bo,
              ln2s, ln2b, w1, b1, w2, b2, kns, knb, hw, hb, out_ref):
    xks = []
    for b in range(2):
        hn, _ = _attn_mlp(
            h_ref[b], ln1s[...], ln1b[...], wq[...], bq[...], wk[...], bk[...],
            wv[...], bv[...], wo[...], bo[...], ln2s[...], ln2b[...],
            w1[...], b1[...], w2[...], b2[...])
        key = _lnv(hn, kns[...], knb[...])
        xks.append(key[0:1, :])
    xk = jnp.concatenate(xks, axis=0)
    xc = jnp.concatenate([xc_ref[b] for b in range(2)], axis=0)
    hw_v = hw[...]
    hb_v = hb[...]
    cl = _dot_t(xc, hw_v) + hb_v
    prob = jax.nn.softmax(cl, axis=-1)
    hw_sum = _dot_t(jnp.ones((1, C), jnp.float32), hw_v)
    assist = prob * hw_sum
    out_ref[...] = _dot_t(xk, hw_v) + hb_v + assist


@functools.cache
def _embed_call():
    return pl.pallas_call(
        _embed_body,
        out_shape=jax.ShapeDtypeStruct((2, 197, C), jnp.float32),
    )


@functools.cache
def _block_call():
    return pl.pallas_call(
        _block_body,
        out_shape=(
            jax.ShapeDtypeStruct((2, 197, C), jnp.float32),
            jax.ShapeDtypeStruct((2, 16, C), jnp.float32),
        ),
    )


@functools.cache
def _clr_call():
    return pl.pallas_call(
        _clr_body,
        out_shape=(
            jax.ShapeDtypeStruct((2, 25, C), jnp.float32),
            jax.ShapeDtypeStruct((2, 1, C), jnp.float32),
        ),
    )


@functools.cache
def _key_call():
    return pl.pallas_call(
        _key_body,
        out_shape=jax.ShapeDtypeStruct((2, 200), jnp.float32),
    )


def kernel(x, w_patch, b_patch, cls_token, pos_emb, ln1_s, ln1_b, wq, bq,
           wk, bk, wv, bv, wo, bo, ln2_s, ln2_b, w1, b1, w2, b2,
           clr_norm_s, clr_norm_b, key_norm_s, key_norm_b, head_w, head_b):
    B = x.shape[0]
    p = x.reshape(B, 3, 14, 16, 14, 16).transpose(0, 2, 4, 1, 3, 5)
    p = p.reshape(B, 196, 3 * 16 * 16)
    h = _embed_call()(p, w_patch, b_patch.reshape(1, C),
                      cls_token.reshape(1, C), pos_emb.reshape(197, C))

    convm = jnp.asarray(_CONV_M)
    r2 = lambda a: a.reshape(1, -1)

    def layer_args(t):
        return (ln1_s[t].reshape(1, C), ln1_b[t].reshape(1, C),
                wq[t], bq[t].reshape(1, C), wk[t], bk[t].reshape(1, C),
                wv[t], bv[t].reshape(1, C), wo[t], bo[t].reshape(1, C),
                ln2_s[t].reshape(1, C), ln2_b[t].reshape(1, C),
                w1[t], b1[t].reshape(1, 3 * C), w2[t], b2[t].reshape(1, C))

    comps = []
    for t in range(11):
        h, gath = _block_call()(h, *layer_args(t), convm)
        comps.append(gath[:, :SEL[t]])

    cls_tok = h[:, 0:1]
    clr_in = jnp.concatenate([cls_tok] + comps, axis=1)
    out25, xc = _clr_call()(clr_in, cls_tok, *layer_args(11),
                            r2(clr_norm_s), r2(clr_norm_b))
    logits = _key_call()(out25, xc, *layer_args(12),
                         r2(key_norm_s), r2(key_norm_b),
                         head_w, head_b.reshape(1, 200))
    return logits


# speedup vs baseline: 1.6015x; 1.6015x over previous
import functools
import math

import numpy as np
import jax
import jax.numpy as jnp
from jax.experimental import pallas as pl

NH = 12
HD = 64
C = 768
SEL = [16, 14, 12, 10, 8, 6, 8, 10, 12, 14, 16]
_NEG = -1e30
_HI = jax.lax.Precision.HIGHEST


def _conv_matrix() -> np.ndarray:
    hs = 14
    k = np.array([[1.0, 2.0, 1.0], [2.0, 4.0, 2.0], [1.0, 2.0, 1.0]], np.float32)
    m = np.zeros((hs * hs, hs * hs), np.float32)
    for r in range(hs):
        for c in range(hs):
            p = r * hs + c
            for dr in (-1, 0, 1):
                for dc in (-1, 0, 1):
                    rr, cc = r + dr, c + dc
                    if 0 <= rr < hs and 0 <= cc < hs:
                        m[rr * hs + cc, p] += k[dr + 1, dc + 1]
    return m


_CONV_M = _conv_matrix()


def _ln(x, s, b, eps=1e-6):
    m = jnp.mean(x, axis=-1, keepdims=True)
    v = jnp.mean((x - m) ** 2, axis=-1, keepdims=True)
    return (x - m) / jnp.sqrt(v + eps) * s + b


def _dot(a, b, prec=None):
    return jnp.dot(a, b, preferred_element_type=jnp.float32, precision=prec)


def _dot_t(a, b, prec=None):
    return jax.lax.dot_general(
        a, b, (((1,), (1,)), ((), ())), preferred_element_type=jnp.float32,
        precision=prec)


def _iter_topk(scores, kk):
    r, p = scores.shape
    lane = jax.lax.broadcasted_iota(jnp.int32, (r, p), 1)
    work = scores
    mask = jnp.zeros((r, p), jnp.float32)
    onehots = []
    for _ in range(kk):
        m = jnp.max(work, axis=1, keepdims=True)
        ismax = work >= m
        idx = jnp.min(jnp.where(ismax, lane, p), axis=1, keepdims=True)
        oh = lane == idx
        ohf = oh.astype(jnp.float32)
        mask = mask + ohf
        onehots.append(ohf)
        work = jnp.where(oh, _NEG, work)
    return mask, onehots


def _embed_body(p_ref, wp_ref, bp_ref, cls_ref, pos_ref, out_ref):
    hp = _dot(p_ref[...], wp_ref[...]) + bp_ref[...]
    for b in range(2):
        out_ref[b * 197:b * 197 + 1, :] = cls_ref[...] + pos_ref[0:1, :]
        out_ref[b * 197 + 1:(b + 1) * 197, :] = (
            hp[b * 196:(b + 1) * 196, :] + pos_ref[1:, :])


def _make_qkv_body(n):
    def body(y_ref, wq, bq, wk, bk, wv, bv, logit_ref, v_ref):
        scale = 1.0 / math.sqrt(HD)
        y = y_ref[...]
        q = _dot(y, wq[...]) + bq[...]
        k = _dot(y, wk[...]) + bk[...]
        v_ref[...] = _dot(y, wv[...]) + bv[...]
        for b in range(2):
            qb = q[b * n:(b + 1) * n, :]
            kb = k[b * n:(b + 1) * n, :]
            for hh in range(NH):
                sl = slice(hh * HD, (hh + 1) * HD)
                logit_ref[b, hh] = _dot_t(qb[:, sl], kb[:, sl]) * scale
    return body


def _make_av_body(n, kk, use_conv):
    def body(att_ref, v_ref, h_ref, wo, bo, convm, h1_ref, g_ref):
        rows = []
        ctxs = []
        for b in range(2):
            vb = v_ref[b * n:(b + 1) * n, :]
            cols = []
            for hh in range(NH):
                sl = slice(hh * HD, (hh + 1) * HD)
                a = att_ref[b, hh]
                rows.append(a[0:1, 1:n])
                cols.append(_dot(a, vb[:, sl]))
            ctxs.append(jnp.concatenate(cols, axis=1))
        ctx = jnp.concatenate(ctxs, axis=0)
        h1_ref[...] = h_ref[...] + (_dot(ctx, wo[...]) + bo[...])
        s_all = jnp.concatenate(rows, axis=0)
        mask, _ = _iter_topk(s_all, 24)
        counts = jnp.concatenate(
            [jnp.sum(mask[b * NH:(b + 1) * NH, :], axis=0, keepdims=True)
             for b in range(2)], axis=0)
        if use_conv:
            counts = _dot(counts, convm[...], _HI)
        _, ohs = _iter_topk(counts, kk)
        for b in range(2):
            g_ref[b] = jnp.concatenate([oh[b:b + 1, :] for oh in ohs], axis=0)
    return body


def _make_av_plain_body(n):
    def body(att_ref, v_ref, h_ref, wo, bo, h1_ref):
        ctxs = []
        for b in range(2):
            vb = v_ref[b * n:(b + 1) * n, :]
            cols = []
            for hh in range(NH):
                sl = slice(hh * HD, (hh + 1) * HD)
                cols.append(_dot(att_ref[b, hh], vb[:, sl]))
            ctxs.append(jnp.concatenate(cols, axis=1))
        ctx = jnp.concatenate(ctxs, axis=0)
        h1_ref[...] = h_ref[...] + (_dot(ctx, wo[...]) + bo[...])
    return body


def _make_mlp_gather_body(n):
    def body(h1_ref, y2_ref, w1, b1, w2, b2, g_ref, h2_ref, gath_ref):
        g = jax.nn.gelu(_dot(y2_ref[...], w1[...]) + b1[...])
        h2 = h1_ref[...] + (_dot(g, w2[...]) + b2[...])
        h2_ref[...] = h2
        for b in range(2):
            gath_ref[b] = _dot(g_ref[b], h2[b * n + 1:(b + 1) * n, :], _HI)
    return body


def _mlp_body(h1_ref, y2_ref, w1, b1, w2, b2, h2_ref):
    g = jax.nn.gelu(_dot(y2_ref[...], w1[...]) + b1[...])
    h2_ref[...] = h1_ref[...] + (_dot(g, w2[...]) + b2[...])


def _make_gather_body(n):
    def body(src_ref, g_ref, out_ref):
        for b in range(2):
            out_ref[b] = _dot(g_ref[b], src_ref[b * n + 1:(b + 1) * n, :], _HI)
    return body


def _head_body(xc_ref, xk_ref, hw, hb, out_ref):
    xc = jnp.concatenate([xc_ref[0], xc_ref[1]], axis=0)
    xk = jnp.concatenate([xk_ref[0], xk_ref[1]], axis=0)
    hw_v = hw[...]
    hb_v = hb[...]
    cl = _dot_t(xc, hw_v) + hb_v
    prob = jax.nn.softmax(cl, axis=-1)
    hw_sum = _dot_t(jnp.ones((1, C), jnp.float32), hw_v, _HI)
    assist = prob * hw_sum
    out_ref[...] = _dot_t(xk, hw_v) + hb_v + assist


@functools.cache
def _embed_call():
    return pl.pallas_call(
        _embed_body, out_shape=jax.ShapeDtypeStruct((394, C), jnp.float32))


@functools.cache
def _qkv_call(n):
    return pl.pallas_call(
        _make_qkv_body(n),
        out_shape=(jax.ShapeDtypeStruct((2, NH, n, n), jnp.float32),
                   jax.ShapeDtypeStruct((2 * n, C), jnp.float32)))


@functools.cache
def _av_vote_call(n, kk, use_conv):
    return pl.pallas_call(
        _make_av_body(n, kk, use_conv),
        out_shape=(jax.ShapeDtypeStruct((2 * n, C), jnp.float32),
                   jax.ShapeDtypeStruct((2, kk, n - 1), jnp.float32)))


@functools.cache
def _av_plain_call(n):
    return pl.pallas_call(
        _make_av_plain_body(n),
        out_shape=jax.ShapeDtypeStruct((2 * n, C), jnp.float32))


@functools.cache
def _mlp_gather_call(n, kk):
    return pl.pallas_call(
        _make_mlp_gather_body(n),
        out_shape=(jax.ShapeDtypeStruct((2 * n, C), jnp.float32),
                   jax.ShapeDtypeStruct((2, kk, C), jnp.float32)))


@functools.cache
def _mlp_call(n):
    return pl.pallas_call(
        _mlp_body, out_shape=jax.ShapeDtypeStruct((2 * n, C), jnp.float32))


@functools.cache
def _gather_call(n, kk):
    return pl.pallas_call(
        _make_gather_body(n),
        out_shape=jax.ShapeDtypeStruct((2, kk, C), jnp.float32))


@functools.cache
def _head_call():
    return pl.pallas_call(
        _head_body, out_shape=jax.ShapeDtypeStruct((2, 200), jnp.float32))


def kernel(x, w_patch, b_patch, cls_token, pos_emb, ln1_s, ln1_b, wq, bq,
           wk, bk, wv, bv, wo, bo, ln2_s, ln2_b, w1, b1, w2, b2,
           clr_norm_s, clr_norm_b, key_norm_s, key_norm_b, head_w, head_b):
    B = x.shape[0]
    p = x.reshape(B, 3, 14, 16, 14, 16).transpose(0, 2, 4, 1, 3, 5)
    p = p.reshape(B * 196, 3 * 16 * 16)
    h = _embed_call()(p, w_patch, b_patch.reshape(1, C),
                      cls_token.reshape(1, C), pos_emb.reshape(197, C))

    convm = jnp.asarray(_CONV_M)
    r2 = lambda a: a.reshape(1, -1)

    def run_block(h, t, n, vote):
        y = _ln(h.reshape(2, n, C), ln1_s[t], ln1_b[t]).reshape(2 * n, C)
        logits, v = _qkv_call(n)(y, wq[t], r2(bq[t]), wk[t], r2(bk[t]),
                                 wv[t], r2(bv[t]))
        att = jax.nn.softmax(logits, axis=-1)
        if vote is None:
            h1 = _av_plain_call(n)(att, v, h, wo[t], r2(bo[t]))
            g = None
        else:
            kk, use_conv = vote
            h1, g = _av_vote_call(n, kk, use_conv)(att, v, h, wo[t],
                                                   r2(bo[t]), convm)
        y2 = _ln(h1.reshape(2, n, C), ln2_s[t], ln2_b[t]).reshape(2 * n, C)
        return h1, y2, g

    comps = []
    for t in range(11):
        h1, y2, g = run_block(h, t, 197, (16, True))
        h, gath = _mlp_gather_call(197, 16)(h1, y2, w1[t], r2(b1[t]),
                                            w2[t], r2(b2[t]), g)
        comps.append(gath[:, :SEL[t]])

    cls_tok = h.reshape(2, 197, C)[:, 0:1]
    clr_in = jnp.concatenate([cls_tok] + comps, axis=1)

    h1, y2, g11 = run_block(clr_in.reshape(254, C), 11, 127, (24, False))
    h2 = _mlp_call(127)(h1, y2, w1[11], r2(b1[11]), w2[11], r2(b2[11]))
    clr = _ln(h2.reshape(2, 127, C), clr_norm_s, clr_norm_b).reshape(254, C)
    gath24 = _gather_call(127, 24)(clr, g11)
    out25 = jnp.concatenate([cls_tok, gath24], axis=1)

    h1, y2, _ = run_block(out25.reshape(50, C), 12, 25, None)
    h2 = _mlp_call(25)(h1, y2, w1[12], r2(b1[12]), w2[12], r2(b2[12]))
    key = _ln(h2.reshape(2, 25, C), key_norm_s, key_norm_b).reshape(50, C)

    xc = clr.reshape(2, 127, C)[:, 0:1]
    xk = key.reshape(2, 25, C)[:, 0:1]
    return _head_call()(xc, xk, head_w, head_b.reshape(1, 200))
